# baseline (speedup 1.0000x reference)
import functools

import jax
import jax.numpy as jnp
from jax import lax
from jax.experimental import pallas as pl
from jax.experimental.pallas import tpu as pltpu
from jax.experimental.pallas import tpu_sc as plsc

N = 10000
E = 320000
D = 128
HID = 64

NC = 2
NS = 16
NW = NC * NS
CH = 128
NCHUNK = E // CH
KBASE = NCHUNK // NW
KREM = NCHUNK - NW * KBASE
KMAX = KBASE + 1
AGG_ROWS = 10240
ROWS_PER_TILE = AGG_ROWS // NS
NBUF = 8
PD = 4
NROUND = -(-KMAX // NBUF)


_hp = functools.partial(jnp.dot, preferred_element_type=jnp.float32)
_hpt = functools.partial(lax.dot_general,
                         dimension_numbers=(((1,), (1,)), ((), ())),
                         preferred_element_type=jnp.float32)


_bf = jnp.bfloat16


def _mlp(x, w1t, b1, w2, b2, w3, b3):
    h = jnp.maximum(_hpt(x, w1t[...].astype(_bf)) + b1[...], 0.0)
    h = jnp.maximum(_hp(h.astype(_bf), w2[...].astype(_bf)) + b2[...], 0.0)
    return _hp(h.astype(_bf), w3[...].astype(_bf)) + b3[...]


def _dense_a1_body(x_ref, w1t, b1, w2, b2, w3, b3, wlt, g_ref):
    x = x_ref[...].astype(_bf)
    h = _mlp(x, w1t, b1, w2, b2, w3, b3)
    g = _hpt(h.astype(_bf), wlt[...].astype(_bf))
    g_ref[...] = jnp.concatenate([g, jnp.zeros_like(g)], axis=1)


def _dense_a2_body(x_ref, w1t, b1, w2, b2, w3, b3, w0t, wxt, bd, d_ref):
    x = x_ref[...].astype(_bf)
    h = _mlp(x, w1t, b1, w2, b2, w3, b3)
    d_ref[...] = (_hpt(h.astype(_bf), w0t[...].astype(_bf))
                  + _hpt(x, wxt[...].astype(_bf)) + bd[...])


def _final_body(p_ref, d_ref, o_ref):
    p = p_ref[0, :, :HID] + p_ref[1, :, :HID]
    o_ref[...] = jnp.maximum(p + d_ref[...], 0.0).T


def _sc_scatter_body(g_hbm, idx_hbm, out_hbm,
                     eb, rows_v, agg_sh, gsem, ssem):
    cid = lax.axis_index("c")
    sid = lax.axis_index("s")
    wid = cid * NS + sid
    kw = KBASE + jnp.where(wid >= NW - KREM, 1, 0)
    start = KBASE * wid + jnp.maximum(wid - (NW - KREM), 0)

    zero16 = jnp.zeros((16,), jnp.float32)

    def zbody(i, c):
        for j in range(HID // 16):
            rows_v[0, i, pl.ds(j * 16, 16)] = zero16
        return c

    lax.fori_loop(0, CH, zbody, 0)
    base = sid * ROWS_PER_TILE
    for t in range(ROWS_PER_TILE // CH):
        pltpu.sync_copy(rows_v.at[0], agg_sh.at[pl.ds(base + t * CH, CH)])

    pltpu.sync_copy(idx_hbm.at[0, pl.ds(start, KMAX)], eb.at[pl.ds(0, KMAX)])
    pltpu.sync_copy(idx_hbm.at[1, pl.ds(start, KMAX)],
                    eb.at[pl.ds(KMAX, KMAX)])

    def dbl(r, c):
        for q in range(CH // 16):
            v = eb[KMAX + r, pl.ds(q * 16, 16)]
            eb[KMAX + r, pl.ds(q * 16, 16)] = v + v
        return c

    lax.fori_loop(0, KMAX, dbl, 0)
    plsc.subcore_barrier()

    def gather(j, bj):
        return pltpu.async_copy(g_hbm.at[eb.at[KMAX + j]], rows_v.at[bj],
                                gsem.at[bj])

    def scatter(i, b):
        return pltpu.async_copy(rows_v.at[b], agg_sh.at[eb.at[i]],
                                ssem.at[b], add=True)

    for b in range(PD):
        gather(b, b)

    def round_body(g, c):
        for b in range(NBUF):
            i = g * NBUF + b
            j = i + PD
            bj = (b + PD) % NBUF

            @pl.when(i < kw)
            def _():
                pltpu.make_async_copy(g_hbm.at[eb.at[KMAX + i]],
                                      rows_v.at[b], gsem.at[b]).wait()
                scatter(i, b)

            @pl.when(jnp.logical_and(j >= NBUF, j < kw))
            def _():
                pltpu.make_async_copy(rows_v.at[bj], agg_sh.at[eb.at[0]],
                                      ssem.at[bj]).wait()

            @pl.when(j < kw)
            def _():
                gather(j, bj)
        return c

    lax.fori_loop(0, NROUND, round_body, 0)
    for b in range(NBUF):
        pltpu.make_async_copy(rows_v.at[b], agg_sh.at[eb.at[0]],
                              ssem.at[b]).wait()
    plsc.subcore_barrier()

    pltpu.sync_copy(agg_sh.at[pl.ds(base, ROWS_PER_TILE)],
                    out_hbm.at[cid, pl.ds(base, ROWS_PER_TILE),
                               pl.ds(0, HID)])


_sc_scatter = pl.kernel(
    _sc_scatter_body,
    out_type=jax.ShapeDtypeStruct((NC, AGG_ROWS, 2 * HID), jnp.float32),
    mesh=plsc.VectorSubcoreMesh(core_axis_name="c", subcore_axis_name="s"),
    scratch_types=[
        pltpu.VMEM((2 * KMAX, CH), jnp.int32),
        pltpu.VMEM((NBUF, CH, HID), jnp.float32),
        pltpu.VMEM_SHARED((AGG_ROWS, HID), jnp.float32),
        pltpu.SemaphoreType.DMA((NBUF,)),
        pltpu.SemaphoreType.DMA((NBUF,)),
    ],
    compiler_params=pltpu.CompilerParams(use_tc_tiling_on_sc=False),
)


def kernel(x, edge_index, edge_type, W1, b1, W2, b2, W3, b3,
           Wl, bl, W0, b0, Wx, bx):
    BR = 1000
    grid = (N // BR,)
    full = lambda shape: pl.BlockSpec(shape, lambda i: (0,) * len(shape))
    bd = (bl + b0 + bx).reshape(1, HID)

    g128 = pl.pallas_call(
        _dense_a1_body,
        grid=grid,
        in_specs=[
            pl.BlockSpec((BR, D), lambda i: (i, 0)),
            full((HID, D)), full((1, HID)),
            full((HID, HID)), full((1, HID)),
            full((HID, D)), full((1, D)),
            full((HID, D)),
        ],
        out_specs=pl.BlockSpec((BR, 2 * HID), lambda i: (i, 0)),
        out_shape=jax.ShapeDtypeStruct((N, 2 * HID), jnp.float32),
    )(x, W1.T, b1.reshape(1, HID), W2, b2.reshape(1, HID),
      W3, b3.reshape(1, D), Wl.T)

    partials = _sc_scatter(g128.reshape(2 * N, HID),
                           edge_index[1].reshape(2, NCHUNK, CH))

    d = pl.pallas_call(
        _dense_a2_body,
        grid=grid,
        in_specs=[
            pl.BlockSpec((BR, D), lambda i: (i, 0)),
            full((HID, D)), full((1, HID)),
            full((HID, HID)), full((1, HID)),
            full((HID, D)), full((1, D)),
            full((HID, D)), full((HID, D)), full((1, HID)),
        ],
        out_specs=pl.BlockSpec((BR, HID), lambda i: (i, 0)),
        out_shape=jax.ShapeDtypeStruct((N, HID), jnp.float32),
    )(x, W1.T, b1.reshape(1, HID), W2, b2.reshape(1, HID),
      W3, b3.reshape(1, D), W0.T, Wx.T, bd)

    out_t = pl.pallas_call(
        _final_body,
        grid=(1,),
        in_specs=[
            pl.BlockSpec((NC, N, 2 * HID), lambda i: (0, 0, 0)),
            pl.BlockSpec((N, HID), lambda i: (0, 0)),
        ],
        out_specs=pl.BlockSpec((HID, N), lambda i: (0, 0)),
        out_shape=jax.ShapeDtypeStruct((HID, N), jnp.float32),
    )(partials, d)
    return out_t.T


# speedup vs baseline: 1.0003x; 1.0003x over previous
import functools

import jax
import jax.numpy as jnp
from jax import lax
from jax.experimental import pallas as pl
from jax.experimental.pallas import tpu as pltpu
from jax.experimental.pallas import tpu_sc as plsc

N = 10000
E = 320000
D = 128
HID = 64

NC = 2
NS = 16
NW = NC * NS
CH = 128
NCHUNK = E // CH
KBASE = NCHUNK // NW
KREM = NCHUNK - NW * KBASE
KMAX = KBASE + 1
AGG_ROWS = 10240
ROWS_PER_TILE = AGG_ROWS // NS
NBUF = 8
PD = 4
NROUND = -(-KMAX // NBUF)


_hp = functools.partial(jnp.dot, preferred_element_type=jnp.float32)
_hpt = functools.partial(lax.dot_general,
                         dimension_numbers=(((1,), (1,)), ((), ())),
                         preferred_element_type=jnp.float32)


def _mlp(x, w1t, b1, w2, b2, w3, b3):
    h = jnp.maximum(_hpt(x, w1t[...]) + b1[...], 0.0)
    h = jnp.maximum(_hp(h, w2[...]) + b2[...], 0.0)
    return _hp(h, w3[...]) + b3[...]


def _dense_a1_body(x_ref, w1t, b1, w2, b2, w3, b3, wlt, g_ref):
    x = x_ref[...]
    h = _mlp(x, w1t, b1, w2, b2, w3, b3)
    g = _hpt(h, wlt[...])
    g_ref[...] = jnp.concatenate([g, jnp.zeros_like(g)], axis=1)


def _dense_a2_body(x_ref, w1t, b1, w2, b2, w3, b3, w0t, wxt, bd, d_ref):
    x = x_ref[...]
    h = _mlp(x, w1t, b1, w2, b2, w3, b3)
    d_ref[...] = _hpt(h, w0t[...]) + _hpt(x, wxt[...]) + bd[...]


def _final_body(p_ref, d_ref, o_ref):
    p = p_ref[0, :, :HID] + p_ref[1, :, :HID]
    o_ref[...] = jnp.maximum(p + d_ref[...], 0.0).T


def _sc_scatter_body(g_hbm, idx_hbm, out_hbm,
                     eb, rows_v, agg_sh, gsem, ssem):
    cid = lax.axis_index("c")
    sid = lax.axis_index("s")
    wid = cid * NS + sid
    kw = KBASE + jnp.where(wid >= NW - KREM, 1, 0)
    start = KBASE * wid + jnp.maximum(wid - (NW - KREM), 0)

    zero16 = jnp.zeros((16,), jnp.float32)

    def zbody(i, c):
        for j in range(HID // 16):
            rows_v[0, i, pl.ds(j * 16, 16)] = zero16
        return c

    lax.fori_loop(0, CH, zbody, 0)
    base = sid * ROWS_PER_TILE
    for t in range(ROWS_PER_TILE // CH):
        pltpu.sync_copy(rows_v.at[0], agg_sh.at[pl.ds(base + t * CH, CH)])

    pltpu.sync_copy(idx_hbm.at[0, pl.ds(start, KMAX)], eb.at[pl.ds(0, KMAX)])
    pltpu.sync_copy(idx_hbm.at[1, pl.ds(start, KMAX)],
                    eb.at[pl.ds(KMAX, KMAX)])

    def dbl(r, c):
        for q in range(CH // 16):
            v = eb[KMAX + r, pl.ds(q * 16, 16)]
            eb[KMAX + r, pl.ds(q * 16, 16)] = v + v
        return c

    lax.fori_loop(0, KMAX, dbl, 0)
    plsc.subcore_barrier()

    def gather(j, bj):
        return pltpu.async_copy(g_hbm.at[eb.at[KMAX + j]], rows_v.at[bj],
                                gsem.at[bj])

    def scatter(i, b):
        return pltpu.async_copy(rows_v.at[b], agg_sh.at[eb.at[i]],
                                ssem.at[b], add=True)

    for b in range(PD):
        gather(b, b)

    def round_body(g, c):
        for b in range(NBUF):
            i = g * NBUF + b
            j = i + PD
            bj = (b + PD) % NBUF

            @pl.when(i < kw)
            def _():
                pltpu.make_async_copy(g_hbm.at[eb.at[KMAX + i]],
                                      rows_v.at[b], gsem.at[b]).wait()
                scatter(i, b)

            @pl.when(jnp.logical_and(j >= NBUF, j < kw))
            def _():
                pltpu.make_async_copy(rows_v.at[bj], agg_sh.at[eb.at[0]],
                                      ssem.at[bj]).wait()

            @pl.when(j < kw)
            def _():
                gather(j, bj)
        return c

    lax.fori_loop(0, NROUND, round_body, 0)
    for b in range(NBUF):
        pltpu.make_async_copy(rows_v.at[b], agg_sh.at[eb.at[0]],
                              ssem.at[b]).wait()
    plsc.subcore_barrier()

    pltpu.sync_copy(agg_sh.at[pl.ds(base, ROWS_PER_TILE)],
                    out_hbm.at[cid, pl.ds(base, ROWS_PER_TILE),
                               pl.ds(0, HID)])


_sc_scatter = pl.kernel(
    _sc_scatter_body,
    out_type=jax.ShapeDtypeStruct((NC, AGG_ROWS, 2 * HID), jnp.float32),
    mesh=plsc.VectorSubcoreMesh(core_axis_name="c", subcore_axis_name="s"),
    scratch_types=[
        pltpu.VMEM((2 * KMAX, CH), jnp.int32),
        pltpu.VMEM((NBUF, CH, HID), jnp.float32),
        pltpu.VMEM_SHARED((AGG_ROWS, HID), jnp.float32),
        pltpu.SemaphoreType.DMA((NBUF,)),
        pltpu.SemaphoreType.DMA((NBUF,)),
    ],
    compiler_params=pltpu.CompilerParams(use_tc_tiling_on_sc=False),
)


def kernel(x, edge_index, edge_type, W1, b1, W2, b2, W3, b3,
           Wl, bl, W0, b0, Wx, bx):
    BR = 1000
    grid = (N // BR,)
    full = lambda shape: pl.BlockSpec(shape, lambda i: (0,) * len(shape))
    bd = (bl + b0 + bx).reshape(1, HID)

    g128 = pl.pallas_call(
        _dense_a1_body,
        grid=grid,
        in_specs=[
            pl.BlockSpec((BR, D), lambda i: (i, 0)),
            full((HID, D)), full((1, HID)),
            full((HID, HID)), full((1, HID)),
            full((HID, D)), full((1, D)),
            full((HID, D)),
        ],
        out_specs=pl.BlockSpec((BR, 2 * HID), lambda i: (i, 0)),
        out_shape=jax.ShapeDtypeStruct((N, 2 * HID), jnp.float32),
    )(x, W1.T, b1.reshape(1, HID), W2, b2.reshape(1, HID),
      W3, b3.reshape(1, D), Wl.T)

    partials = _sc_scatter(g128.reshape(2 * N, HID),
                           edge_index[1].reshape(2, NCHUNK, CH))

    d = pl.pallas_call(
        _dense_a2_body,
        grid=grid,
        in_specs=[
            pl.BlockSpec((BR, D), lambda i: (i, 0)),
            full((HID, D)), full((1, HID)),
            full((HID, HID)), full((1, HID)),
            full((HID, D)), full((1, D)),
            full((HID, D)), full((HID, D)), full((1, HID)),
        ],
        out_specs=pl.BlockSpec((BR, HID), lambda i: (i, 0)),
        out_shape=jax.ShapeDtypeStruct((N, HID), jnp.float32),
    )(x, W1.T, b1.reshape(1, HID), W2, b2.reshape(1, HID),
      W3, b3.reshape(1, D), W0.T, Wx.T, bd)

    out_t = pl.pallas_call(
        _final_body,
        grid=(1,),
        in_specs=[
            pl.BlockSpec((NC, N, 2 * HID), lambda i: (0, 0, 0)),
            pl.BlockSpec((N, HID), lambda i: (0, 0)),
        ],
        out_specs=pl.BlockSpec((HID, N), lambda i: (0, 0)),
        out_shape=jax.ShapeDtypeStruct((HID, N), jnp.float32),
    )(partials, d)
    return out_t.T


# speedup vs baseline: 1.0018x; 1.0015x over previous
import functools

import jax
import jax.numpy as jnp
from jax import lax
from jax.experimental import pallas as pl
from jax.experimental.pallas import tpu as pltpu
from jax.experimental.pallas import tpu_sc as plsc

N = 10000
E = 320000
D = 128
HID = 64

NC = 2
NS = 16
NW = NC * NS
CH = 128
NCHUNK = E // CH
KBASE = NCHUNK // NW
KREM = NCHUNK - NW * KBASE
KMAX = KBASE + 1
AGG_ROWS = 10240
ROWS_PER_TILE = AGG_ROWS // NS
NBUF = 8
PD = 4
NROUND = -(-KMAX // NBUF)


_hp = functools.partial(jnp.dot, preferred_element_type=jnp.float32)
_hpt = functools.partial(lax.dot_general,
                         dimension_numbers=(((1,), (1,)), ((), ())),
                         preferred_element_type=jnp.float32)


def _mlp(x, w1t, b1, w2, b2, w3, b3):
    h = jnp.maximum(_hpt(x, w1t[...]) + b1[...], 0.0)
    h = jnp.maximum(_hp(h, w2[...]) + b2[...], 0.0)
    return _hp(h, w3[...]) + b3[...]


def _dense_a1_body(x_ref, w1t, b1, w2, b2, w3, b3, wlt, g_ref):
    x = x_ref[...]
    h = _mlp(x, w1t, b1, w2, b2, w3, b3)
    g = _hpt(h, wlt[...])
    g_ref[...] = jnp.concatenate([g, jnp.zeros_like(g)], axis=1)


def _dense_a2_body(x_ref, w1t, b1, w2, b2, w3, b3, w0t, wxt, bd, d_ref):
    x = x_ref[...]
    h = _mlp(x, w1t, b1, w2, b2, w3, b3)
    d_ref[...] = _hpt(h, w0t[...]) + _hpt(x, wxt[...]) + bd[...]


def _final_body(p_ref, d_ref, o_ref):
    p = p_ref[:, :HID] + p_ref[:, HID:]
    o_ref[...] = jnp.maximum(p + d_ref[...], 0.0).T


def _sc_scatter_body(g_hbm, idx_hbm, out_hbm,
                     eb, rows_v, agg_sh, gsem, ssem):
    cid = lax.axis_index("c")
    sid = lax.axis_index("s")
    wid = cid * NS + sid
    kw = KBASE + jnp.where(wid >= NW - KREM, 1, 0)
    start = KBASE * wid + jnp.maximum(wid - (NW - KREM), 0)

    zero16 = jnp.zeros((16,), jnp.float32)

    def zbody(i, c):
        for j in range(HID // 16):
            rows_v[0, i, pl.ds(j * 16, 16)] = zero16
        return c

    lax.fori_loop(0, CH, zbody, 0)
    base = sid * ROWS_PER_TILE
    for t in range(ROWS_PER_TILE // CH):
        pltpu.sync_copy(rows_v.at[0], agg_sh.at[pl.ds(base + t * CH, CH)])

    pltpu.sync_copy(idx_hbm.at[0, pl.ds(start, KMAX)], eb.at[pl.ds(0, KMAX)])
    pltpu.sync_copy(idx_hbm.at[1, pl.ds(start, KMAX)],
                    eb.at[pl.ds(KMAX, KMAX)])

    def dbl(r, c):
        for q in range(CH // 16):
            v = eb[KMAX + r, pl.ds(q * 16, 16)]
            eb[KMAX + r, pl.ds(q * 16, 16)] = v + v
        return c

    lax.fori_loop(0, KMAX, dbl, 0)
    plsc.subcore_barrier()

    def gather(j, bj):
        return pltpu.async_copy(g_hbm.at[eb.at[KMAX + j]], rows_v.at[bj],
                                gsem.at[bj])

    def scatter(i, b):
        return pltpu.async_copy(rows_v.at[b], agg_sh.at[eb.at[i]],
                                ssem.at[b], add=True)

    for b in range(PD):
        gather(b, b)

    def round_body(g, c):
        for b in range(NBUF):
            i = g * NBUF + b
            j = i + PD
            bj = (b + PD) % NBUF

            @pl.when(i < kw)
            def _():
                pltpu.make_async_copy(g_hbm.at[eb.at[KMAX + i]],
                                      rows_v.at[b], gsem.at[b]).wait()
                scatter(i, b)

            @pl.when(jnp.logical_and(j >= NBUF, j < kw))
            def _():
                pltpu.make_async_copy(rows_v.at[bj], agg_sh.at[eb.at[0]],
                                      ssem.at[bj]).wait()

            @pl.when(j < kw)
            def _():
                gather(j, bj)
        return c

    lax.fori_loop(0, NROUND, round_body, 0)
    for b in range(NBUF):
        pltpu.make_async_copy(rows_v.at[b], agg_sh.at[eb.at[0]],
                              ssem.at[b]).wait()
    plsc.subcore_barrier()

    pltpu.sync_copy(agg_sh.at[pl.ds(base, ROWS_PER_TILE)],
                    out_hbm.at[pl.ds(base, ROWS_PER_TILE),
                               pl.ds(cid * HID, HID)])


_sc_scatter = pl.kernel(
    _sc_scatter_body,
    out_type=jax.ShapeDtypeStruct((AGG_ROWS, 2 * HID), jnp.float32),
    mesh=plsc.VectorSubcoreMesh(core_axis_name="c", subcore_axis_name="s"),
    scratch_types=[
        pltpu.VMEM((2 * KMAX, CH), jnp.int32),
        pltpu.VMEM((NBUF, CH, HID), jnp.float32),
        pltpu.VMEM_SHARED((AGG_ROWS, HID), jnp.float32),
        pltpu.SemaphoreType.DMA((NBUF,)),
        pltpu.SemaphoreType.DMA((NBUF,)),
    ],
    compiler_params=pltpu.CompilerParams(use_tc_tiling_on_sc=False),
)


def kernel(x, edge_index, edge_type, W1, b1, W2, b2, W3, b3,
           Wl, bl, W0, b0, Wx, bx):
    BR = 1000
    grid = (N // BR,)
    full = lambda shape: pl.BlockSpec(shape, lambda i: (0,) * len(shape))
    bd = (bl + b0 + bx).reshape(1, HID)

    g128 = pl.pallas_call(
        _dense_a1_body,
        grid=grid,
        in_specs=[
            pl.BlockSpec((BR, D), lambda i: (i, 0)),
            full((HID, D)), full((1, HID)),
            full((HID, HID)), full((1, HID)),
            full((HID, D)), full((1, D)),
            full((HID, D)),
        ],
        out_specs=pl.BlockSpec((BR, 2 * HID), lambda i: (i, 0)),
        out_shape=jax.ShapeDtypeStruct((N, 2 * HID), jnp.float32),
    )(x, W1.T, b1.reshape(1, HID), W2, b2.reshape(1, HID),
      W3, b3.reshape(1, D), Wl.T)

    partials = _sc_scatter(g128.reshape(2 * N, HID),
                           edge_index[1].reshape(2, NCHUNK, CH))

    d = pl.pallas_call(
        _dense_a2_body,
        grid=grid,
        in_specs=[
            pl.BlockSpec((BR, D), lambda i: (i, 0)),
            full((HID, D)), full((1, HID)),
            full((HID, HID)), full((1, HID)),
            full((HID, D)), full((1, D)),
            full((HID, D)), full((HID, D)), full((1, HID)),
        ],
        out_specs=pl.BlockSpec((BR, HID), lambda i: (i, 0)),
        out_shape=jax.ShapeDtypeStruct((N, HID), jnp.float32),
    )(x, W1.T, b1.reshape(1, HID), W2, b2.reshape(1, HID),
      W3, b3.reshape(1, D), W0.T, Wx.T, bd)

    out_t = pl.pallas_call(
        _final_body,
        grid=(1,),
        in_specs=[
            pl.BlockSpec((N, 2 * HID), lambda i: (0, 0)),
            pl.BlockSpec((N, HID), lambda i: (0, 0)),
        ],
        out_specs=pl.BlockSpec((HID, N), lambda i: (0, 0)),
        out_shape=jax.ShapeDtypeStruct((HID, N), jnp.float32),
    )(partials, d)
    return out_t.T


# speedup vs baseline: 1.0387x; 1.0368x over previous
import functools

import jax
import jax.numpy as jnp
from jax import lax
from jax.experimental import pallas as pl
from jax.experimental.pallas import tpu as pltpu
from jax.experimental.pallas import tpu_sc as plsc

N = 10000
E = 320000
D = 128
HID = 64

NC = 2
NS = 16
NW = NC * NS
CH = 128
NCHUNK = E // CH
KBASE = NCHUNK // NW
KREM = NCHUNK - NW * KBASE
KMAX = KBASE + 1
AGG_ROWS = 10240
ROWS_PER_TILE = AGG_ROWS // NS
NBUF = 8
PD = 4
NROUND = -(-KMAX // NBUF)


_hp = functools.partial(jnp.dot, preferred_element_type=jnp.float32)
_hpt = functools.partial(lax.dot_general,
                         dimension_numbers=(((1,), (1,)), ((), ())),
                         preferred_element_type=jnp.float32)


def _mlp(x, w1t, b1, w2, b2, w3, b3):
    h = jnp.maximum(_hpt(x, w1t[...]) + b1[...], 0.0)
    h = jnp.maximum(_hp(h, w2[...]) + b2[...], 0.0)
    return _hp(h, w3[...]) + b3[...]


def _dense_a1_body(x_ref, w1t, b1, w2, b2, w3, b3, wlt, g_ref):
    x = x_ref[...]
    h = _mlp(x, w1t, b1, w2, b2, w3, b3)
    g = _hpt(h, wlt[...])
    g_ref[...] = jnp.concatenate([g, jnp.zeros_like(g)], axis=1)


def _dense_a2_body(x_ref, w1t, b1, w2, b2, w3, b3, w0t, wxt, bd, d_ref):
    x = x_ref[...]
    h = _mlp(x, w1t, b1, w2, b2, w3, b3)
    d_ref[...] = _hpt(h, w0t[...]) + _hpt(x, wxt[...]) + bd[...]


def _final_body(p_ref, d_ref, o_ref):
    p = p_ref[:, :HID] + p_ref[:, HID:]
    o_ref[...] = jnp.maximum(p + d_ref[...], 0.0).T


def _sc_scatter_body(g_hbm, idx_hbm, out_hbm,
                     eb, rows_v, agg_sh, gsem, ssem):
    cid = lax.axis_index("c")
    sid = lax.axis_index("s")
    wid = cid * NS + sid
    kw = KBASE + jnp.where(wid >= NW - KREM, 1, 0)
    start = KBASE * wid + jnp.maximum(wid - (NW - KREM), 0)

    zero16 = jnp.zeros((16,), jnp.float32)

    def zbody(i, c):
        for j in range(HID // 16):
            rows_v[0, i, pl.ds(j * 16, 16)] = zero16
        return c

    lax.fori_loop(0, CH, zbody, 0)
    base = sid * ROWS_PER_TILE
    for t in range(ROWS_PER_TILE // CH):
        pltpu.sync_copy(rows_v.at[0], agg_sh.at[pl.ds(base + t * CH, CH)])

    pltpu.sync_copy(idx_hbm.at[0, pl.ds(start, KMAX)], eb.at[pl.ds(0, KMAX)])
    pltpu.sync_copy(idx_hbm.at[1, pl.ds(start, KMAX)],
                    eb.at[pl.ds(KMAX, KMAX)])

    def dbl(r, c):
        for q in range(CH // 16):
            v = eb[KMAX + r, pl.ds(q * 16, 16)]
            eb[KMAX + r, pl.ds(q * 16, 16)] = v + v
        return c

    lax.fori_loop(0, KMAX, dbl, 0)
    plsc.subcore_barrier()

    def gather(j, bj):
        return pltpu.async_copy(g_hbm.at[eb.at[KMAX + j]], rows_v.at[bj],
                                gsem.at[bj])

    def scatter(i, b):
        return pltpu.async_copy(rows_v.at[b], agg_sh.at[eb.at[i]],
                                ssem.at[b], add=True)

    for b in range(PD):
        gather(b, b)

    def round_body(g, c):
        for b in range(NBUF):
            i = g * NBUF + b
            j = i + PD
            bj = (b + PD) % NBUF

            @pl.when(i < kw)
            def _():
                pltpu.make_async_copy(g_hbm.at[eb.at[KMAX + i]],
                                      rows_v.at[b], gsem.at[b]).wait()
                scatter(i, b)

            @pl.when(jnp.logical_and(j >= NBUF, j < kw))
            def _():
                pltpu.make_async_copy(rows_v.at[bj], agg_sh.at[eb.at[0]],
                                      ssem.at[bj]).wait()

            @pl.when(j < kw)
            def _():
                gather(j, bj)
        return c

    lax.fori_loop(0, NROUND, round_body, 0)
    for b in range(NBUF):
        pltpu.make_async_copy(rows_v.at[b], agg_sh.at[eb.at[0]],
                              ssem.at[b]).wait()
    plsc.subcore_barrier()

    pltpu.sync_copy(agg_sh.at[pl.ds(base, ROWS_PER_TILE)],
                    out_hbm.at[pl.ds(base, ROWS_PER_TILE),
                               pl.ds(cid * HID, HID)])


_sc_scatter = pl.kernel(
    _sc_scatter_body,
    out_type=jax.ShapeDtypeStruct((AGG_ROWS, 2 * HID), jnp.float32),
    mesh=plsc.VectorSubcoreMesh(core_axis_name="c", subcore_axis_name="s"),
    scratch_types=[
        pltpu.VMEM((2 * KMAX, CH), jnp.int32),
        pltpu.VMEM((NBUF, CH, HID), jnp.float32),
        pltpu.VMEM_SHARED((AGG_ROWS, HID), jnp.float32),
        pltpu.SemaphoreType.DMA((NBUF,)),
        pltpu.SemaphoreType.DMA((NBUF,)),
    ],
    compiler_params=pltpu.CompilerParams(use_tc_tiling_on_sc=False),
)


def kernel(x, edge_index, edge_type, W1, b1, W2, b2, W3, b3,
           Wl, bl, W0, b0, Wx, bx):
    BR = 2000
    grid = (N // BR,)
    full = lambda shape: pl.BlockSpec(shape, lambda i: (0,) * len(shape))
    bd = (bl + b0 + bx).reshape(1, HID)

    g128 = pl.pallas_call(
        _dense_a1_body,
        grid=grid,
        in_specs=[
            pl.BlockSpec((BR, D), lambda i: (i, 0)),
            full((HID, D)), full((1, HID)),
            full((HID, HID)), full((1, HID)),
            full((HID, D)), full((1, D)),
            full((HID, D)),
        ],
        out_specs=pl.BlockSpec((BR, 2 * HID), lambda i: (i, 0)),
        out_shape=jax.ShapeDtypeStruct((N, 2 * HID), jnp.float32),
    )(x, W1.T, b1.reshape(1, HID), W2, b2.reshape(1, HID),
      W3, b3.reshape(1, D), Wl.T)

    partials = _sc_scatter(g128.reshape(2 * N, HID),
                           edge_index[1].reshape(2, NCHUNK, CH))

    d = pl.pallas_call(
        _dense_a2_body,
        grid=grid,
        in_specs=[
            pl.BlockSpec((BR, D), lambda i: (i, 0)),
            full((HID, D)), full((1, HID)),
            full((HID, HID)), full((1, HID)),
            full((HID, D)), full((1, D)),
            full((HID, D)), full((HID, D)), full((1, HID)),
        ],
        out_specs=pl.BlockSpec((BR, HID), lambda i: (i, 0)),
        out_shape=jax.ShapeDtypeStruct((N, HID), jnp.float32),
    )(x, W1.T, b1.reshape(1, HID), W2, b2.reshape(1, HID),
      W3, b3.reshape(1, D), W0.T, Wx.T, bd)

    out_t = pl.pallas_call(
        _final_body,
        grid=(1,),
        in_specs=[
            pl.BlockSpec((N, 2 * HID), lambda i: (0, 0)),
            pl.BlockSpec((N, HID), lambda i: (0, 0)),
        ],
        out_specs=pl.BlockSpec((HID, N), lambda i: (0, 0)),
        out_shape=jax.ShapeDtypeStruct((HID, N), jnp.float32),
    )(partials, d)
    return out_t.T


# speedup vs baseline: 1.1000x; 1.0591x over previous
import functools

import jax
import jax.numpy as jnp
from jax import lax
from jax.experimental import pallas as pl
from jax.experimental.pallas import tpu as pltpu
from jax.experimental.pallas import tpu_sc as plsc

N = 10000
E = 320000
D = 128
HID = 64

NC = 2
NS = 16
NW = NC * NS
CH = 128
NCHUNK = E // CH
KBASE = NCHUNK // NW
KREM = NCHUNK - NW * KBASE
KMAX = KBASE + 1
AGG_ROWS = 10240
ROWS_PER_TILE = AGG_ROWS // NS
NBUF = 8
PD = 6
NROUND = -(-KMAX // NBUF)


_hp = functools.partial(jnp.dot, preferred_element_type=jnp.float32)
_hpt = functools.partial(lax.dot_general,
                         dimension_numbers=(((1,), (1,)), ((), ())),
                         preferred_element_type=jnp.float32)


def _mlp(x, w1t, b1, w2, b2, w3, b3):
    h = jnp.maximum(_hpt(x, w1t[...]) + b1[...], 0.0)
    h = jnp.maximum(_hp(h, w2[...]) + b2[...], 0.0)
    return _hp(h, w3[...]) + b3[...]


def _dense_a1_body(x_ref, w1t, b1, w2, b2, w3, b3, wlt, g_ref):
    x = x_ref[...]
    h = _mlp(x, w1t, b1, w2, b2, w3, b3)
    g = _hpt(h, wlt[...])
    g_ref[...] = jnp.concatenate([g, jnp.zeros_like(g)], axis=1)


def _dense_a2_body(x_ref, w1t, b1, w2, b2, w3, b3, w0t, wxt, bd, d_ref):
    x = x_ref[...]
    h = _mlp(x, w1t, b1, w2, b2, w3, b3)
    d_ref[...] = _hpt(h, w0t[...]) + _hpt(x, wxt[...]) + bd[...]


def _final_body(p_ref, d_ref, o_ref):
    p = p_ref[:, :HID] + p_ref[:, HID:]
    o_ref[...] = jnp.maximum(p + d_ref[...], 0.0).T


def _sc_scatter_body(g_hbm, idx_hbm, out_hbm,
                     eb, rows_v, agg_sh, gsem, ssem):
    cid = lax.axis_index("c")
    sid = lax.axis_index("s")
    wid = cid * NS + sid
    kw = KBASE + jnp.where(wid >= NW - KREM, 1, 0)
    start = KBASE * wid + jnp.maximum(wid - (NW - KREM), 0)

    zero16 = jnp.zeros((16,), jnp.float32)

    def zbody(i, c):
        for j in range(HID // 16):
            rows_v[0, i, pl.ds(j * 16, 16)] = zero16
        return c

    lax.fori_loop(0, CH, zbody, 0)
    base = sid * ROWS_PER_TILE
    for t in range(ROWS_PER_TILE // CH):
        pltpu.sync_copy(rows_v.at[0], agg_sh.at[pl.ds(base + t * CH, CH)])

    pltpu.sync_copy(idx_hbm.at[0, pl.ds(start, KMAX)], eb.at[pl.ds(0, KMAX)])
    pltpu.sync_copy(idx_hbm.at[1, pl.ds(start, KMAX)],
                    eb.at[pl.ds(KMAX, KMAX)])

    def dbl(r, c):
        for q in range(CH // 16):
            v = eb[KMAX + r, pl.ds(q * 16, 16)]
            eb[KMAX + r, pl.ds(q * 16, 16)] = v + v
        return c

    lax.fori_loop(0, KMAX, dbl, 0)
    plsc.subcore_barrier()

    def gather(j, bj):
        return pltpu.async_copy(g_hbm.at[eb.at[KMAX + j]], rows_v.at[bj],
                                gsem.at[bj])

    def scatter(i, b):
        return pltpu.async_copy(rows_v.at[b], agg_sh.at[eb.at[i]],
                                ssem.at[b], add=True)

    for b in range(PD):
        gather(b, b)

    def round_body(g, c):
        for b in range(NBUF):
            i = g * NBUF + b
            j = i + PD
            bj = (b + PD) % NBUF

            @pl.when(i < kw)
            def _():
                pltpu.make_async_copy(g_hbm.at[eb.at[KMAX + i]],
                                      rows_v.at[b], gsem.at[b]).wait()
                scatter(i, b)

            @pl.when(jnp.logical_and(j >= NBUF, j < kw))
            def _():
                pltpu.make_async_copy(rows_v.at[bj], agg_sh.at[eb.at[0]],
                                      ssem.at[bj]).wait()

            @pl.when(j < kw)
            def _():
                gather(j, bj)
        return c

    lax.fori_loop(0, NROUND, round_body, 0)
    for b in range(NBUF):
        pltpu.make_async_copy(rows_v.at[b], agg_sh.at[eb.at[0]],
                              ssem.at[b]).wait()
    plsc.subcore_barrier()

    pltpu.sync_copy(agg_sh.at[pl.ds(base, ROWS_PER_TILE)],
                    out_hbm.at[pl.ds(base, ROWS_PER_TILE),
                               pl.ds(cid * HID, HID)])


_sc_scatter = pl.kernel(
    _sc_scatter_body,
    out_type=jax.ShapeDtypeStruct((AGG_ROWS, 2 * HID), jnp.float32),
    mesh=plsc.VectorSubcoreMesh(core_axis_name="c", subcore_axis_name="s"),
    scratch_types=[
        pltpu.VMEM((2 * KMAX, CH), jnp.int32),
        pltpu.VMEM((NBUF, CH, HID), jnp.float32),
        pltpu.VMEM_SHARED((AGG_ROWS, HID), jnp.float32),
        pltpu.SemaphoreType.DMA((NBUF,)),
        pltpu.SemaphoreType.DMA((NBUF,)),
    ],
    compiler_params=pltpu.CompilerParams(use_tc_tiling_on_sc=False),
)


def kernel(x, edge_index, edge_type, W1, b1, W2, b2, W3, b3,
           Wl, bl, W0, b0, Wx, bx):
    BR = 2000
    grid = (N // BR,)
    full = lambda shape: pl.BlockSpec(shape, lambda i: (0,) * len(shape))
    bd = (bl + b0 + bx).reshape(1, HID)

    g128 = pl.pallas_call(
        _dense_a1_body,
        grid=grid,
        in_specs=[
            pl.BlockSpec((BR, D), lambda i: (i, 0)),
            full((HID, D)), full((1, HID)),
            full((HID, HID)), full((1, HID)),
            full((HID, D)), full((1, D)),
            full((HID, D)),
        ],
        out_specs=pl.BlockSpec((BR, 2 * HID), lambda i: (i, 0)),
        out_shape=jax.ShapeDtypeStruct((N, 2 * HID), jnp.float32),
    )(x, W1.T, b1.reshape(1, HID), W2, b2.reshape(1, HID),
      W3, b3.reshape(1, D), Wl.T)

    partials = _sc_scatter(g128.reshape(2 * N, HID),
                           edge_index[1].reshape(2, NCHUNK, CH))

    d = pl.pallas_call(
        _dense_a2_body,
        grid=grid,
        in_specs=[
            pl.BlockSpec((BR, D), lambda i: (i, 0)),
            full((HID, D)), full((1, HID)),
            full((HID, HID)), full((1, HID)),
            full((HID, D)), full((1, D)),
            full((HID, D)), full((HID, D)), full((1, HID)),
        ],
        out_specs=pl.BlockSpec((BR, HID), lambda i: (i, 0)),
        out_shape=jax.ShapeDtypeStruct((N, HID), jnp.float32),
    )(x, W1.T, b1.reshape(1, HID), W2, b2.reshape(1, HID),
      W3, b3.reshape(1, D), W0.T, Wx.T, bd)

    out_t = pl.pallas_call(
        _final_body,
        grid=(1,),
        in_specs=[
            pl.BlockSpec((N, 2 * HID), lambda i: (0, 0)),
            pl.BlockSpec((N, HID), lambda i: (0, 0)),
        ],
        out_specs=pl.BlockSpec((HID, N), lambda i: (0, 0)),
        out_shape=jax.ShapeDtypeStruct((HID, N), jnp.float32),
    )(partials, d)
    return out_t.T


# speedup vs baseline: 1.1005x; 1.0005x over previous
import functools

import jax
import jax.numpy as jnp
from jax import lax
from jax.experimental import pallas as pl
from jax.experimental.pallas import tpu as pltpu
from jax.experimental.pallas import tpu_sc as plsc

N = 10000
E = 320000
D = 128
HID = 64

NC = 2
NS = 16
NW = NC * NS
CH = 128
NCHUNK = E // CH
KBASE = NCHUNK // NW
KREM = NCHUNK - NW * KBASE
KMAX = KBASE + 1
AGG_ROWS = 10240
ROWS_PER_TILE = AGG_ROWS // NS
NBUF = 8
PD = 7
NROUND = -(-KMAX // NBUF)


_hp = functools.partial(jnp.dot, preferred_element_type=jnp.float32)
_hpt = functools.partial(lax.dot_general,
                         dimension_numbers=(((1,), (1,)), ((), ())),
                         preferred_element_type=jnp.float32)


def _mlp(x, w1t, b1, w2, b2, w3, b3):
    h = jnp.maximum(_hpt(x, w1t[...]) + b1[...], 0.0)
    h = jnp.maximum(_hp(h, w2[...]) + b2[...], 0.0)
    return _hp(h, w3[...]) + b3[...]


def _dense_a1_body(x_ref, w1t, b1, w2, b2, w3, b3, wlt, g_ref):
    x = x_ref[...]
    h = _mlp(x, w1t, b1, w2, b2, w3, b3)
    g = _hpt(h, wlt[...])
    g_ref[...] = jnp.concatenate([g, jnp.zeros_like(g)], axis=1)


def _dense_a2_body(x_ref, w1t, b1, w2, b2, w3, b3, w0t, wxt, bd, d_ref):
    x = x_ref[...]
    h = _mlp(x, w1t, b1, w2, b2, w3, b3)
    d_ref[...] = _hpt(h, w0t[...]) + _hpt(x, wxt[...]) + bd[...]


def _final_body(p_ref, d_ref, o_ref):
    p = p_ref[:, :HID] + p_ref[:, HID:]
    o_ref[...] = jnp.maximum(p + d_ref[...], 0.0).T


def _sc_scatter_body(g_hbm, idx_hbm, out_hbm,
                     eb, rows_v, agg_sh, gsem, ssem):
    cid = lax.axis_index("c")
    sid = lax.axis_index("s")
    wid = cid * NS + sid
    kw = KBASE + jnp.where(wid >= NW - KREM, 1, 0)
    start = KBASE * wid + jnp.maximum(wid - (NW - KREM), 0)

    zero16 = jnp.zeros((16,), jnp.float32)

    def zbody(i, c):
        for j in range(HID // 16):
            rows_v[0, i, pl.ds(j * 16, 16)] = zero16
        return c

    lax.fori_loop(0, CH, zbody, 0)
    base = sid * ROWS_PER_TILE
    for t in range(ROWS_PER_TILE // CH):
        pltpu.sync_copy(rows_v.at[0], agg_sh.at[pl.ds(base + t * CH, CH)])

    pltpu.sync_copy(idx_hbm.at[0, pl.ds(start, KMAX)], eb.at[pl.ds(0, KMAX)])
    pltpu.sync_copy(idx_hbm.at[1, pl.ds(start, KMAX)],
                    eb.at[pl.ds(KMAX, KMAX)])

    def dbl(r, c):
        for q in range(CH // 16):
            v = eb[KMAX + r, pl.ds(q * 16, 16)]
            eb[KMAX + r, pl.ds(q * 16, 16)] = v + v
        return c

    lax.fori_loop(0, KMAX, dbl, 0)
    plsc.subcore_barrier()

    def gather(j, bj):
        return pltpu.async_copy(g_hbm.at[eb.at[KMAX + j]], rows_v.at[bj],
                                gsem.at[bj])

    def scatter(i, b):
        return pltpu.async_copy(rows_v.at[b], agg_sh.at[eb.at[i]],
                                ssem.at[b], add=True)

    for b in range(PD):
        gather(b, b)

    def round_body(g, c):
        for b in range(NBUF):
            i = g * NBUF + b
            j = i + PD
            bj = (b + PD) % NBUF

            @pl.when(i < kw)
            def _():
                pltpu.make_async_copy(g_hbm.at[eb.at[KMAX + i]],
                                      rows_v.at[b], gsem.at[b]).wait()
                scatter(i, b)

            @pl.when(jnp.logical_and(j >= NBUF, j < kw))
            def _():
                pltpu.make_async_copy(rows_v.at[bj], agg_sh.at[eb.at[0]],
                                      ssem.at[bj]).wait()

            @pl.when(j < kw)
            def _():
                gather(j, bj)
        return c

    lax.fori_loop(0, NROUND, round_body, 0)
    for b in range(NBUF):
        pltpu.make_async_copy(rows_v.at[b], agg_sh.at[eb.at[0]],
                              ssem.at[b]).wait()
    plsc.subcore_barrier()

    pltpu.sync_copy(agg_sh.at[pl.ds(base, ROWS_PER_TILE)],
                    out_hbm.at[pl.ds(base, ROWS_PER_TILE),
                               pl.ds(cid * HID, HID)])


_sc_scatter = pl.kernel(
    _sc_scatter_body,
    out_type=jax.ShapeDtypeStruct((AGG_ROWS, 2 * HID), jnp.float32),
    mesh=plsc.VectorSubcoreMesh(core_axis_name="c", subcore_axis_name="s"),
    scratch_types=[
        pltpu.VMEM((2 * KMAX, CH), jnp.int32),
        pltpu.VMEM((NBUF, CH, HID), jnp.float32),
        pltpu.VMEM_SHARED((AGG_ROWS, HID), jnp.float32),
        pltpu.SemaphoreType.DMA((NBUF,)),
        pltpu.SemaphoreType.DMA((NBUF,)),
    ],
    compiler_params=pltpu.CompilerParams(use_tc_tiling_on_sc=False),
)


def kernel(x, edge_index, edge_type, W1, b1, W2, b2, W3, b3,
           Wl, bl, W0, b0, Wx, bx):
    BR = 2000
    grid = (N // BR,)
    full = lambda shape: pl.BlockSpec(shape, lambda i: (0,) * len(shape))
    bd = (bl + b0 + bx).reshape(1, HID)

    g128 = pl.pallas_call(
        _dense_a1_body,
        grid=grid,
        in_specs=[
            pl.BlockSpec((BR, D), lambda i: (i, 0)),
            full((HID, D)), full((1, HID)),
            full((HID, HID)), full((1, HID)),
            full((HID, D)), full((1, D)),
            full((HID, D)),
        ],
        out_specs=pl.BlockSpec((BR, 2 * HID), lambda i: (i, 0)),
        out_shape=jax.ShapeDtypeStruct((N, 2 * HID), jnp.float32),
    )(x, W1.T, b1.reshape(1, HID), W2, b2.reshape(1, HID),
      W3, b3.reshape(1, D), Wl.T)

    partials = _sc_scatter(g128.reshape(2 * N, HID),
                           edge_index[1].reshape(2, NCHUNK, CH))

    d = pl.pallas_call(
        _dense_a2_body,
        grid=grid,
        in_specs=[
            pl.BlockSpec((BR, D), lambda i: (i, 0)),
            full((HID, D)), full((1, HID)),
            full((HID, HID)), full((1, HID)),
            full((HID, D)), full((1, D)),
            full((HID, D)), full((HID, D)), full((1, HID)),
        ],
        out_specs=pl.BlockSpec((BR, HID), lambda i: (i, 0)),
        out_shape=jax.ShapeDtypeStruct((N, HID), jnp.float32),
    )(x, W1.T, b1.reshape(1, HID), W2, b2.reshape(1, HID),
      W3, b3.reshape(1, D), W0.T, Wx.T, bd)

    out_t = pl.pallas_call(
        _final_body,
        grid=(1,),
        in_specs=[
            pl.BlockSpec((N, 2 * HID), lambda i: (0, 0)),
            pl.BlockSpec((N, HID), lambda i: (0, 0)),
        ],
        out_specs=pl.BlockSpec((HID, N), lambda i: (0, 0)),
        out_shape=jax.ShapeDtypeStruct((HID, N), jnp.float32),
    )(partials, d)
    return out_t.T
